# baseline (speedup 1.0000x reference)
import functools

import jax
import jax.numpy as jnp
from jax import lax
from jax.experimental import pallas as pl
from jax.experimental.pallas import tpu as pltpu
from jax.experimental.pallas import tpu_sc as plsc

VOCAB = 30522
_VPAD = 30720
D = 768
_NV = D // 16
EPS = 1e-12


_ROWS_BLK = 2048


def _stats_body(x_ref, mu_ref, rstd_ref):
    x = x_ref[...]
    mu = jnp.mean(x, axis=-1)
    xc = x - mu[:, None]
    var = jnp.mean(xc * xc, axis=-1)
    mu_ref[...] = mu
    rstd_ref[...] = lax.rsqrt(var + EPS)


def _stats_table(word_table):
    n_blocks = pl.cdiv(VOCAB, _ROWS_BLK)
    return pl.pallas_call(
        _stats_body,
        grid=(n_blocks,),
        in_specs=[pl.BlockSpec((_ROWS_BLK, D), lambda i: (i, 0))],
        out_specs=[pl.BlockSpec((_ROWS_BLK,), lambda i: (i,)),
                   pl.BlockSpec((_ROWS_BLK,), lambda i: (i,))],
        out_shape=[jax.ShapeDtypeStruct((_VPAD,), jnp.float32),
                   jax.ShapeDtypeStruct((_VPAD,), jnp.float32)],
    )(word_table)


_info = plsc.get_sparse_core_info()
_NC, _NS = _info.num_cores, _info.num_subcores
_NW = _NC * _NS

N_TOK = 128 * 512
_PER_W = N_TOK // _NW
_CH = 16
_NBUF = 4
_NCHUNK = _PER_W // _CH
_NGRP = _NCHUNK // _NBUF

_mesh = plsc.VectorSubcoreMesh(core_axis_name="c", subcore_axis_name="s")


@functools.partial(
    pl.kernel,
    mesh=_mesh,
    compiler_params=pltpu.CompilerParams(needs_layout_passes=False),
    out_type=jax.ShapeDtypeStruct((N_TOK, D), jnp.float32),
    scratch_types=[
        pltpu.VMEM((_PER_W,), jnp.int32),
        pltpu.VMEM((_CH, D), jnp.float32),
        pltpu.VMEM((_CH, D), jnp.float32),
        pltpu.VMEM((_CH, D), jnp.float32),
        pltpu.VMEM((_CH, D), jnp.float32),
        pltpu.VMEM((_VPAD,), jnp.float32),
        pltpu.VMEM((_VPAD,), jnp.float32),
        pltpu.VMEM((D,), jnp.float32),
        pltpu.VMEM((D,), jnp.float32),
        pltpu.SemaphoreType.DMA,
        pltpu.SemaphoreType.DMA,
        pltpu.SemaphoreType.DMA,
        pltpu.SemaphoreType.DMA,
        pltpu.SemaphoreType.DMA,
        pltpu.SemaphoreType.DMA,
        pltpu.SemaphoreType.DMA,
        pltpu.SemaphoreType.DMA,
    ],
)
def _sc_apply(table_hbm, mu_hbm, rstd_hbm, idx_hbm, gamma_hbm, beta_hbm,
              out_hbm, idx_v, b0, b1, b2, b3, mu_v, rstd_v, gv, bv,
              sg0, sg1, sg2, sg3, ss0, ss1, ss2, ss3):
    wid = lax.axis_index("s") * _NC + lax.axis_index("c")
    base = wid * _PER_W
    pltpu.sync_copy(idx_hbm.at[pl.ds(base, _PER_W)], idx_v)
    pltpu.sync_copy(mu_hbm, mu_v)
    pltpu.sync_copy(rstd_hbm, rstd_v)
    pltpu.sync_copy(gamma_hbm, gv)
    pltpu.sync_copy(beta_hbm, bv)

    bufs = (b0, b1, b2, b3)
    sgs = (sg0, sg1, sg2, sg3)
    sss = (ss0, ss1, ss2, ss3)

    def issue_gather(c, b):
        pltpu.async_copy(table_hbm.at[idx_v.at[pl.ds(c * _CH, _CH)]],
                         bufs[b], sgs[b])

    def wait_gather(c, b):
        pltpu.make_async_copy(table_hbm.at[idx_v.at[pl.ds(c * _CH, _CH)]],
                              bufs[b], sgs[b]).wait()

    def issue_scatter(c, b):
        pltpu.async_copy(bufs[b], out_hbm.at[pl.ds(base + c * _CH, _CH)],
                         sss[b])

    def wait_scatter(c, b):
        pltpu.make_async_copy(bufs[b], out_hbm.at[pl.ds(base + c * _CH, _CH)],
                              sss[b]).wait()

    dnums = lax.GatherDimensionNumbers(
        offset_dims=(), collapsed_slice_dims=(0,), start_index_map=(0,))

    def shuffle(x, perm):
        return lax.gather(x, perm[:, None], dnums, slice_sizes=(1,),
                          mode=lax.GatherScatterMode.PROMISE_IN_BOUNDS)

    def apply_chunk(c, rb):
        ivec = idx_v[pl.ds(c * _CH, _CH)]
        mu16 = plsc.load_gather(mu_v, [ivec])
        rs16 = plsc.load_gather(rstd_v, [ivec])

        mus = []
        rss = []
        for r in range(_CH):
            perm = jnp.full((16,), r, dtype=jnp.int32)
            mus.append(shuffle(mu16, perm))
            rss.append(shuffle(rs16, perm))

        @plsc.parallel_loop(0, _NV, 1, unroll=2, carry=(mus, rss))
        def jbody(j, carry):
            cmus, crss = carry
            sl = pl.ds(16 * j, 16)
            g = gv[sl]
            bb = bv[sl]
            for r in range(_CH):
                x = rb[r, sl]
                rb[r, sl] = (x - cmus[r]) * crss[r] * g + bb
            return carry

    issue_gather(0, 0)
    issue_gather(1, 1)

    def group(t, carry):
        for b in range(_NBUF):
            c = t * _NBUF + b
            wait_gather(c, b)
            b2 = (b + 2) % _NBUF

            @pl.when(jnp.logical_and(c + 2 < _NCHUNK, c - 2 >= 0))
            def _():
                wait_scatter(c - 2, b2)
                issue_gather(c + 2, b2)

            @pl.when(c - 2 < 0)
            def _():
                issue_gather(c + 2, b2)

            apply_chunk(c, bufs[b])
            issue_scatter(c, b)
        return carry

    lax.fori_loop(0, _NGRP, group, 0, unroll=False)
    wait_scatter(_NCHUNK - 2, 2)
    wait_scatter(_NCHUNK - 1, 3)


def kernel(input_ids, token_type_ids, position_ids, word_table, pos_table,
           tt_table, ln_gamma, ln_beta):
    del token_type_ids, position_ids, pos_table, tt_table
    mu, rstd = _stats_table(word_table)
    ids_flat = input_ids.reshape(N_TOK).astype(jnp.int32)
    out = _sc_apply(word_table, mu, rstd, ids_flat, ln_gamma, ln_beta)
    B, S = input_ids.shape
    return out.reshape(B, S, D)


# speedup vs baseline: 1.7846x; 1.7846x over previous
import functools

import jax
import jax.numpy as jnp
from jax import lax
from jax.experimental import pallas as pl
from jax.experimental.pallas import tpu as pltpu
from jax.experimental.pallas import tpu_sc as plsc

VOCAB = 30522
D = 768
EPS = 1e-12


_ROWS_BLK = 2048


def _ln_body(x_ref, g_ref, b_ref, o_ref):
    x = x_ref[...]
    mu = jnp.mean(x, axis=-1, keepdims=True)
    xc = x - mu
    var = jnp.mean(xc * xc, axis=-1, keepdims=True)
    o_ref[...] = (xc * lax.rsqrt(var + EPS)) * g_ref[...] + b_ref[...]


def _normalize_table(word_table, ln_gamma, ln_beta):
    n_blocks = pl.cdiv(VOCAB, _ROWS_BLK)
    return pl.pallas_call(
        _ln_body,
        grid=(n_blocks,),
        in_specs=[
            pl.BlockSpec((_ROWS_BLK, D), lambda i: (i, 0)),
            pl.BlockSpec((1, D), lambda i: (0, 0)),
            pl.BlockSpec((1, D), lambda i: (0, 0)),
        ],
        out_specs=pl.BlockSpec((_ROWS_BLK, D), lambda i: (i, 0)),
        out_shape=jax.ShapeDtypeStruct((VOCAB, D), jnp.float32),
    )(word_table, ln_gamma.reshape(1, D), ln_beta.reshape(1, D))


_info = plsc.get_sparse_core_info()
_NC, _NS = _info.num_cores, _info.num_subcores
_NW = _NC * _NS

N_TOK = 128 * 512
_PER_W = N_TOK // _NW
_CH = 16
_NBUF = 8
_PF = 4
_NCHUNK = _PER_W // _CH
_NGRP = _NCHUNK // _NBUF

_mesh = plsc.VectorSubcoreMesh(core_axis_name="c", subcore_axis_name="s")


@functools.partial(
    pl.kernel,
    mesh=_mesh,
    out_type=jax.ShapeDtypeStruct((N_TOK, D), jnp.float32),
    scratch_types=[
        pltpu.VMEM((_PER_W,), jnp.int32),
    ] + [pltpu.VMEM((_CH, D), jnp.float32)] * 8
      + [pltpu.SemaphoreType.DMA] * 16,
)
def _sc_gather(table_hbm, idx_hbm, out_hbm, idx_v,
               b0, b1, b2, b3, b4, b5, b6, b7,
               sg0, sg1, sg2, sg3, sg4, sg5, sg6, sg7,
               ss0, ss1, ss2, ss3, ss4, ss5, ss6, ss7):
    wid = lax.axis_index("s") * _NC + lax.axis_index("c")
    base = wid * _PER_W
    pltpu.sync_copy(idx_hbm.at[pl.ds(base, _PER_W)], idx_v)

    bufs = (b0, b1, b2, b3, b4, b5, b6, b7)
    sgs = (sg0, sg1, sg2, sg3, sg4, sg5, sg6, sg7)
    sss = (ss0, ss1, ss2, ss3, ss4, ss5, ss6, ss7)

    def issue_gather(c, b):
        pltpu.async_copy(table_hbm.at[idx_v.at[pl.ds(c * _CH, _CH)]],
                         bufs[b], sgs[b])

    def wait_gather(c, b):
        pltpu.make_async_copy(table_hbm.at[idx_v.at[pl.ds(c * _CH, _CH)]],
                              bufs[b], sgs[b]).wait()

    def issue_scatter(c, b):
        pltpu.async_copy(bufs[b], out_hbm.at[pl.ds(base + c * _CH, _CH)],
                         sss[b])

    def wait_scatter(c, b):
        pltpu.make_async_copy(bufs[b], out_hbm.at[pl.ds(base + c * _CH, _CH)],
                              sss[b]).wait()

    for c in range(_PF):
        issue_gather(c, c)

    def group(t, carry):
        for b in range(_NBUF):
            c = t * _NBUF + b
            wait_gather(c, b)
            issue_scatter(c, b)
            cp = c + _PF
            bp = (b + _PF) % _NBUF

            @pl.when(jnp.logical_and(cp < _NCHUNK, cp - _NBUF >= 0))
            def _():
                wait_scatter(cp - _NBUF, bp)
                issue_gather(cp, bp)

            @pl.when(jnp.logical_and(cp < _NCHUNK, cp - _NBUF < 0))
            def _():
                issue_gather(cp, bp)
        return carry

    lax.fori_loop(0, _NGRP, group, 0, unroll=False)
    for i in range(_NBUF):
        c = _NCHUNK - _NBUF + i
        wait_scatter(c, c % _NBUF)


def kernel(input_ids, token_type_ids, position_ids, word_table, pos_table,
           tt_table, ln_gamma, ln_beta):
    del token_type_ids, position_ids, pos_table, tt_table
    normed = _normalize_table(word_table, ln_gamma, ln_beta)
    ids_flat = input_ids.reshape(N_TOK).astype(jnp.int32)
    out = _sc_gather(normed, ids_flat)
    B, S = input_ids.shape
    return out.reshape(B, S, D)
